# baseline (speedup 1.0000x reference)
import functools
import jax
import jax.numpy as jnp
from jax import lax
from jax.experimental import pallas as pl
from jax.experimental.pallas import tpu as pltpu
from jax.experimental.pallas import tpu_sc as plsc

_EPS = 1e-8
_NC = 2
_NS = 16
_LB = 128


def _ceil_to(x, m):
    return (x + m - 1) // m * m


def _make_scatter(epadh, nv, half):
    mesh = plsc.VectorSubcoreMesh(core_axis_name="c", subcore_axis_name="s")
    _G = 2
    ew = epadh // _NS
    nbs = ew // _LB
    ng = nbs // _G
    np_h = _ceil_to(half + 64, _NS * 8)
    zr = np_h // _NS // 8
    cr = half // _NS

    @functools.partial(
        pl.kernel,
        out_type=jax.ShapeDtypeStruct((4, nv, 64), jnp.float32),
        mesh=mesh,
        scratch_types=[
            pltpu.VMEM((zr, 64), jnp.float32),
            pltpu.VMEM((nbs, _LB), jnp.int32),
            pltpu.VMEM((nbs, _LB), jnp.int32),
            pltpu.VMEM((2, _G, _LB, 64), jnp.float32),
            pltpu.VMEM_SHARED((np_h, 64), jnp.float32),
            pltpu.SemaphoreType.DMA,
            pltpu.SemaphoreType.DMA,
        ],
        compiler_params=pltpu.CompilerParams(use_tc_tiling_on_sc=False),
    )
    def scatter_kernel(tbl, src4, dst, out, zbuf, sidx, didx, rows, acc,
                       gsem, ssem):
        c = lax.axis_index("c")
        s = lax.axis_index("s")
        zv = jnp.zeros((16,), jnp.float32)
        base = c * half

        @pl.loop(0, zr)
        def _(r):
            for q in range(4):
                zbuf[r, pl.ds(q * 16, 16)] = zv

        for chunk in range(4):
            pltpu.sync_copy(src4.at[chunk, c, pl.ds(s * nbs, nbs)], sidx)
            pltpu.sync_copy(dst.at[chunk, c, pl.ds(s * nbs, nbs)], didx)
            for bb in range(8):
                pltpu.sync_copy(
                    zbuf, acc.at[pl.ds(s * (np_h // _NS) + bb * zr, zr)]
                )
            plsc.subcore_barrier()

            @pl.loop(0, ng // 2)
            def _(p):
                ga = [
                    pltpu.async_copy(tbl.at[sidx.at[(2 * p) * _G + j]],
                                     rows.at[0, j], gsem)
                    for j in range(_G)
                ]
                for d in ga:
                    d.wait()
                sa = [
                    pltpu.async_copy(rows.at[0, j],
                                     acc.at[didx.at[(2 * p) * _G + j]],
                                     ssem, add=True)
                    for j in range(_G)
                ]
                gb = [
                    pltpu.async_copy(tbl.at[sidx.at[(2 * p + 1) * _G + j]],
                                     rows.at[1, j], gsem)
                    for j in range(_G)
                ]
                for d in gb:
                    d.wait()
                for d in sa:
                    d.wait()
                sb = [
                    pltpu.async_copy(rows.at[1, j],
                                     acc.at[didx.at[(2 * p + 1) * _G + j]],
                                     ssem, add=True)
                    for j in range(_G)
                ]
                for d in sb:
                    d.wait()

            plsc.subcore_barrier()

            k = 0
            while k < cr:
                w = min(_LB, cr - k)
                pltpu.sync_copy(acc.at[pl.ds(s * cr + k, w)],
                                rows.at[0, 0, pl.ds(0, w)])
                pltpu.sync_copy(rows.at[0, 0, pl.ds(0, w)],
                                out.at[chunk, pl.ds(base + s * cr + k, w)])
                k += w

            plsc.subcore_barrier()

    return scatter_kernel


def _row_block_specs(bn):
    return pl.BlockSpec((bn, 256), lambda ii: (ii, 0))


def _chunk_block_spec(bn):
    return pl.BlockSpec((4, bn, 64), lambda ii: (0, ii, 0))


def _scale_a(x, cnt, bn):
    n = x.shape[0]

    def body(x_ref, c_ref, a_ref, xa_ref):
        a = 1.0 / (jnp.sqrt(c_ref[...] + 1.0) + _EPS)
        a_ref[...] = a
        xa = x_ref[...] * a
        for c in range(4):
            xa_ref[c] = xa[:, c * 64:(c + 1) * 64]

    return pl.pallas_call(
        body,
        grid=(n // bn,),
        in_specs=[
            _row_block_specs(bn),
            pl.BlockSpec((bn, 1), lambda ii: (ii, 0)),
        ],
        out_specs=[
            pl.BlockSpec((bn, 1), lambda ii: (ii, 0)),
            _chunk_block_spec(bn),
        ],
        out_shape=[
            jax.ShapeDtypeStruct((n, 1), jnp.float32),
            jax.ShapeDtypeStruct((4, n, 64), jnp.float32),
        ],
    )(x, cnt)


def _layer1(s4, xa4, a, w, bn):
    n = s4.shape[1]

    def body(s_ref, p_ref, a_ref, w_ref, h_ref, ha_ref):
        av = a_ref[...]
        acc = jnp.zeros((s_ref.shape[1], 256), jnp.float32)
        for c in range(4):
            m = av * (s_ref[c] + p_ref[c])
            acc = acc + jnp.dot(m, w_ref[...][c * 64:(c + 1) * 64, :],
                                preferred_element_type=jnp.float32)
        h = jnp.tanh(acc)
        h_ref[...] = h
        hv = av * h
        for c in range(4):
            ha_ref[c] = hv[:, c * 64:(c + 1) * 64]

    return pl.pallas_call(
        body,
        grid=(n // bn,),
        in_specs=[
            _chunk_block_spec(bn),
            _chunk_block_spec(bn),
            pl.BlockSpec((bn, 1), lambda ii: (ii, 0)),
            pl.BlockSpec((256, 256), lambda ii: (0, 0)),
        ],
        out_specs=[
            _row_block_specs(bn),
            _chunk_block_spec(bn),
        ],
        out_shape=[
            jax.ShapeDtypeStruct((n, 256), jnp.float32),
            jax.ShapeDtypeStruct((4, n, 64), jnp.float32),
        ],
    )(s4, xa4, a, w)


def _layer2(s4, pa4, a, w, x, h1, bn):
    n = s4.shape[1]

    def body(s_ref, p_ref, a_ref, w_ref, x_ref, h1_ref, o_ref, o4_ref):
        av = a_ref[...]
        acc = jnp.zeros((s_ref.shape[1], 256), jnp.float32)
        for c in range(4):
            m = av * (s_ref[c] + p_ref[c])
            acc = acc + jnp.dot(m, w_ref[...][c * 64:(c + 1) * 64, :],
                                preferred_element_type=jnp.float32)
        o = (x_ref[...] + h1_ref[...] + jnp.tanh(acc)) * (1.0 / 3.0)
        o_ref[...] = o
        for c in range(4):
            o4_ref[c] = o[:, c * 64:(c + 1) * 64]

    return pl.pallas_call(
        body,
        grid=(n // bn,),
        in_specs=[
            _chunk_block_spec(bn),
            _chunk_block_spec(bn),
            pl.BlockSpec((bn, 1), lambda ii: (ii, 0)),
            pl.BlockSpec((256, 256), lambda ii: (0, 0)),
            _row_block_specs(bn),
            _row_block_specs(bn),
        ],
        out_specs=[
            _row_block_specs(bn),
            _chunk_block_spec(bn),
        ],
        out_shape=[
            jax.ShapeDtypeStruct((n, 256), jnp.float32),
            jax.ShapeDtypeStruct((4, n, 64), jnp.float32),
        ],
    )(s4, pa4, a, w, x, h1)


def _gate_users(il, bl, ft, gw, b2, nrows, bn):
    def body(il_ref, bl_ref, ft_ref, gw_ref, b_ref, o_ref):
        ilv = il_ref[...]
        blv = bl_ref[...]
        m = jnp.concatenate([ilv, blv, ft_ref[...]], axis=1)
        g = jnp.dot(m, gw_ref[...], preferred_element_type=jnp.float32)
        g = g + b_ref[...][0:1, :]
        w0 = 1.0 / (1.0 + jnp.exp(g[:, 1:2] - g[:, 0:1]))
        o_ref[...] = w0 * ilv + (1.0 - w0) * blv

    return pl.pallas_call(
        body,
        grid=(nrows // bn,),
        in_specs=[
            _row_block_specs(bn),
            _row_block_specs(bn),
            _row_block_specs(bn),
            pl.BlockSpec((768, 2), lambda ii: (0, 0)),
            pl.BlockSpec((1, 2), lambda ii: (0, 0)),
        ],
        out_specs=_row_block_specs(bn),
        out_shape=jax.ShapeDtypeStruct((nrows, 256), jnp.float32),
    )(il, bl, ft, gw, b2)


def _gate_bundles(il4, cnt, bl, ft, gw, b2, nrows, u, bn):
    def body(il_ref, c_ref, bl_ref, ft_ref, gw_ref, b_ref, o_ref):
        inv = 1.0 / (c_ref[...] + _EPS)
        ilv = jnp.concatenate(
            [il_ref[c] for c in range(4)], axis=1) * inv
        blv = bl_ref[...]
        m = jnp.concatenate([ilv, blv, ft_ref[...]], axis=1)
        g = jnp.dot(m, gw_ref[...], preferred_element_type=jnp.float32)
        g = g + b_ref[...][0:1, :]
        w0 = 1.0 / (1.0 + jnp.exp(g[:, 1:2] - g[:, 0:1]))
        o_ref[...] = w0 * ilv + (1.0 - w0) * blv

    uoff = u // bn
    return pl.pallas_call(
        body,
        grid=(nrows // bn,),
        in_specs=[
            _chunk_block_spec(bn),
            pl.BlockSpec((bn, 1), lambda ii: (ii, 0)),
            pl.BlockSpec((bn, 256), lambda ii: (ii + uoff, 0)),
            _row_block_specs(bn),
            pl.BlockSpec((768, 2), lambda ii: (0, 0)),
            pl.BlockSpec((1, 2), lambda ii: (0, 0)),
        ],
        out_specs=_row_block_specs(bn),
        out_shape=jax.ShapeDtypeStruct((nrows, 256), jnp.float32),
    )(il4, cnt, bl, ft, gw, b2)


def kernel(ui_edge_index, ub_edge_index, bi_edge_index, users_feature,
           items_feature, bundles_feature, W1_item, W2_item, W1_bundle,
           W2_bundle, gate_W, gate_b):
    u = users_feature.shape[0]
    i = items_feature.shape[0]
    b = bundles_feature.shape[0]
    n = u + i
    e = ui_edge_index.shape[1]

    ui0 = ui_edge_index[0].astype(jnp.int32)
    ui1 = ui_edge_index[1].astype(jnp.int32)
    ub0 = ub_edge_index[0].astype(jnp.int32)
    ub1 = ub_edge_index[1].astype(jnp.int32)
    bi0 = bi_edge_index[0].astype(jnp.int32)
    bi1 = bi_edge_index[1].astype(jnp.int32)

    epadh = _ceil_to(e, _NS * _LB * 8)
    nbt = epadh // _LB
    half = u
    coffs2 = (jnp.arange(4, dtype=jnp.int32) * n)[:, None, None]
    empty = jnp.zeros((0,), jnp.int32)
    spread = jnp.arange(epadh, dtype=jnp.int32) % n

    def padl(src, dl):
        ne = src.shape[0]
        pad = jnp.arange(epadh - ne, dtype=jnp.int32)
        return (jnp.concatenate([src, pad % n]),
                jnp.concatenate([dl, half + pad % 64]))

    def edges2(src0, dl0, src1, dl1):
        s0, d0 = padl(src0, dl0)
        s1, d1 = padl(src1, dl1)
        src2 = jnp.stack([s0, s1])
        d2 = jnp.stack([d0, d1])
        dst4 = jnp.broadcast_to(d2[None], (4, 2, epadh)).reshape(
            4, 2, nbt, _LB)
        src43 = (src2[None] + coffs2).reshape(4, 2, nbt, _LB)
        return src43, dst4

    trash = half + spread % 64
    hist_d = [
        (padl(spread[:e], ui0)[1], padl(spread[:e], ui1)[1]),
        (padl(spread[:e], ub0)[1], padl(spread[:e], ub1)[1]),
        (padl(spread[:e], bi0)[1], trash),
        (trash, trash),
    ]
    s4_h = (jnp.broadcast_to(spread[None, None], (4, 2, epadh)) +
            coffs2).reshape(4, 2, nbt, _LB)
    d_h = jnp.stack([jnp.stack(list(p)) for p in hist_d]).reshape(
        4, 2, nbt, _LB)

    s4_ui, d_ui = edges2(ui1 + u, ui0, ui0, ui1)
    s4_ub, d_ub = edges2(ub1 + u, ub0, ub0, ub1)
    s4_bi, d_bi = edges2(bi1 + u, bi0, empty, empty)

    src4_stack = jnp.stack([s4_h, s4_ui, s4_ui, s4_ub, s4_ub, s4_bi])
    dst_stack = jnp.stack([d_h, d_ui, d_ui, d_ub, d_ub, d_bi])

    scat = _make_scatter(epadh, n, half)
    bn = 1000

    x_ui = jnp.concatenate([users_feature, items_feature], axis=0)
    x_ub = jnp.concatenate([users_feature, bundles_feature], axis=0)
    ones_tbl = jnp.ones((4 * n, 64), jnp.float32)

    zn1 = jnp.zeros((n, 1), jnp.float32)
    znd = jnp.zeros((n, 256), jnp.float32)
    zn4 = jnp.zeros((4, n, 64), jnp.float32)
    stash0 = (zn1, zn1, zn1, zn4, zn4, znd, znd, znd, znd, zn4, zn4)

    def _upd(st, **kw):
        names = ["a_ui", "a_ub", "cnt_bi", "xa4_ui", "xa4_ub", "h1_ui",
                 "h1_ub", "out_ui", "out_ub", "out4_ui", "pooled4"]
        lst = list(st)
        for k, v in kw.items():
            lst[names.index(k)] = v
        return tuple(lst)

    def br_hist(tbl, out4, st):
        a_ui, xa4_ui = _scale_a(x_ui, out4[0, :, 0:1], bn)
        a_ub, xa4_ub = _scale_a(x_ub, out4[1, :, 0:1], bn)
        return xa4_ui.reshape(4 * n, 64), _upd(
            st, a_ui=a_ui, a_ub=a_ub, cnt_bi=out4[2, :, 0:1],
            xa4_ui=xa4_ui, xa4_ub=xa4_ub)

    def br_s1_ui(tbl, out4, st):
        h1, ha4 = _layer1(out4, st[3], st[0], W1_item, bn)
        return ha4.reshape(4 * n, 64), _upd(st, h1_ui=h1)

    def br_s2_ui(tbl, out4, st):
        out, o4 = _layer2(out4, tbl.reshape(4, n, 64), st[0], W2_item, x_ui,
                          st[5], bn)
        return st[4].reshape(4 * n, 64), _upd(st, out_ui=out, out4_ui=o4)

    def br_s1_ub(tbl, out4, st):
        h1, ha4 = _layer1(out4, st[4], st[1], W1_bundle, bn)
        return ha4.reshape(4 * n, 64), _upd(st, h1_ub=h1)

    def br_s2_ub(tbl, out4, st):
        out, _o4 = _layer2(out4, tbl.reshape(4, n, 64), st[1], W2_bundle,
                           x_ub, st[6], bn)
        return st[9].reshape(4 * n, 64), _upd(st, out_ub=out)

    def br_pool(tbl, out4, st):
        return tbl, _upd(st, pooled4=out4)

    branches = [br_hist, br_s1_ui, br_s2_ui, br_s1_ub, br_s2_ub, br_pool]

    def body(carry, xs):
        tbl, st = carry
        src4, dst, pid = xs
        out4 = scat(tbl, src4, dst)
        tbl2, st2 = lax.switch(pid, branches, tbl, out4, st)
        return (tbl2, st2), None

    (_, stash), _ = lax.scan(
        body, (ones_tbl, stash0),
        (src4_stack, dst_stack, jnp.arange(6, dtype=jnp.int32)))

    b2 = gate_b.reshape(1, 2)
    users_out = _gate_users(stash[7], stash[8], users_feature, gate_W, b2,
                            u, bn)
    bundles_out = _gate_bundles(stash[10], stash[2], stash[8],
                                bundles_feature, gate_W, b2, b, u, bn)

    return jnp.concatenate([users_out, bundles_out], axis=0)


# speedup vs baseline: 1.0111x; 1.0111x over previous
import functools
import jax
import jax.numpy as jnp
from jax import lax
from jax.experimental import pallas as pl
from jax.experimental.pallas import tpu as pltpu
from jax.experimental.pallas import tpu_sc as plsc

_EPS = 1e-8
_NC = 2
_NS = 16
_LB = 128


def _ceil_to(x, m):
    return (x + m - 1) // m * m


def _make_scatter(epadh, nv, half):
    mesh = plsc.VectorSubcoreMesh(core_axis_name="c", subcore_axis_name="s")
    _G = 5
    ew = epadh // _NS
    nbs = ew // _LB
    ng = nbs // _G
    np_h = _ceil_to(half + 64, _NS * 8)
    zr = np_h // _NS // 8
    cr = half // _NS

    @functools.partial(
        pl.kernel,
        out_type=jax.ShapeDtypeStruct((4, nv, 64), jnp.float32),
        mesh=mesh,
        scratch_types=[
            pltpu.VMEM((zr, 64), jnp.float32),
            pltpu.VMEM((nbs, _LB), jnp.int32),
            pltpu.VMEM((nbs, _LB), jnp.int32),
            pltpu.VMEM((_G, _LB, 64), jnp.float32),
            pltpu.VMEM_SHARED((np_h, 64), jnp.float32),
            pltpu.SemaphoreType.DMA,
            pltpu.SemaphoreType.DMA,
        ],
        compiler_params=pltpu.CompilerParams(use_tc_tiling_on_sc=False),
    )
    def scatter_kernel(tbl, src4, dst, out, zbuf, sidx, didx, rows, acc,
                       gsem, ssem):
        c = lax.axis_index("c")
        s = lax.axis_index("s")
        zv = jnp.zeros((16,), jnp.float32)
        base = c * half

        @pl.loop(0, zr)
        def _(r):
            for q in range(4):
                zbuf[r, pl.ds(q * 16, 16)] = zv

        for chunk in range(4):
            pltpu.sync_copy(src4.at[chunk, c, pl.ds(s * nbs, nbs)], sidx)
            pltpu.sync_copy(dst.at[chunk, c, pl.ds(s * nbs, nbs)], didx)
            for bb in range(8):
                pltpu.sync_copy(
                    zbuf, acc.at[pl.ds(s * (np_h // _NS) + bb * zr, zr)]
                )
            plsc.subcore_barrier()

            @pl.loop(0, ng)
            def _(g):
                gd = [
                    pltpu.async_copy(tbl.at[sidx.at[g * _G + j]],
                                     rows.at[j], gsem)
                    for j in range(_G)
                ]
                for d in gd:
                    d.wait()
                sd = [
                    pltpu.async_copy(rows.at[j], acc.at[didx.at[g * _G + j]],
                                     ssem, add=True)
                    for j in range(_G)
                ]
                for d in sd:
                    d.wait()

            plsc.subcore_barrier()

            k = 0
            while k < cr:
                w = min(_LB, cr - k)
                pltpu.sync_copy(acc.at[pl.ds(s * cr + k, w)],
                                rows.at[0, pl.ds(0, w)])
                pltpu.sync_copy(rows.at[0, pl.ds(0, w)],
                                out.at[chunk, pl.ds(base + s * cr + k, w)])
                k += w

            plsc.subcore_barrier()

    return scatter_kernel


def _row_block_specs(bn):
    return pl.BlockSpec((bn, 256), lambda ii: (ii, 0))


def _chunk_block_spec(bn):
    return pl.BlockSpec((4, bn, 64), lambda ii: (0, ii, 0))


def _scale_a(x, cnt, bn):
    n = x.shape[0]

    def body(x_ref, c_ref, a_ref, xa_ref):
        a = 1.0 / (jnp.sqrt(c_ref[...] + 1.0) + _EPS)
        a_ref[...] = a
        xa = x_ref[...] * a
        for c in range(4):
            xa_ref[c] = xa[:, c * 64:(c + 1) * 64]

    return pl.pallas_call(
        body,
        grid=(n // bn,),
        in_specs=[
            _row_block_specs(bn),
            pl.BlockSpec((bn, 1), lambda ii: (ii, 0)),
        ],
        out_specs=[
            pl.BlockSpec((bn, 1), lambda ii: (ii, 0)),
            _chunk_block_spec(bn),
        ],
        out_shape=[
            jax.ShapeDtypeStruct((n, 1), jnp.float32),
            jax.ShapeDtypeStruct((4, n, 64), jnp.float32),
        ],
    )(x, cnt)


def _layer1(s4, xa4, a, w, bn):
    n = s4.shape[1]

    def body(s_ref, p_ref, a_ref, w_ref, h_ref, ha_ref):
        av = a_ref[...]
        acc = jnp.zeros((s_ref.shape[1], 256), jnp.float32)
        for c in range(4):
            m = av * (s_ref[c] + p_ref[c])
            acc = acc + jnp.dot(m, w_ref[...][c * 64:(c + 1) * 64, :],
                                preferred_element_type=jnp.float32)
        h = jnp.tanh(acc)
        h_ref[...] = h
        hv = av * h
        for c in range(4):
            ha_ref[c] = hv[:, c * 64:(c + 1) * 64]

    return pl.pallas_call(
        body,
        grid=(n // bn,),
        in_specs=[
            _chunk_block_spec(bn),
            _chunk_block_spec(bn),
            pl.BlockSpec((bn, 1), lambda ii: (ii, 0)),
            pl.BlockSpec((256, 256), lambda ii: (0, 0)),
        ],
        out_specs=[
            _row_block_specs(bn),
            _chunk_block_spec(bn),
        ],
        out_shape=[
            jax.ShapeDtypeStruct((n, 256), jnp.float32),
            jax.ShapeDtypeStruct((4, n, 64), jnp.float32),
        ],
    )(s4, xa4, a, w)


def _layer2(s4, pa4, a, w, x, h1, bn):
    n = s4.shape[1]

    def body(s_ref, p_ref, a_ref, w_ref, x_ref, h1_ref, o_ref, o4_ref):
        av = a_ref[...]
        acc = jnp.zeros((s_ref.shape[1], 256), jnp.float32)
        for c in range(4):
            m = av * (s_ref[c] + p_ref[c])
            acc = acc + jnp.dot(m, w_ref[...][c * 64:(c + 1) * 64, :],
                                preferred_element_type=jnp.float32)
        o = (x_ref[...] + h1_ref[...] + jnp.tanh(acc)) * (1.0 / 3.0)
        o_ref[...] = o
        for c in range(4):
            o4_ref[c] = o[:, c * 64:(c + 1) * 64]

    return pl.pallas_call(
        body,
        grid=(n // bn,),
        in_specs=[
            _chunk_block_spec(bn),
            _chunk_block_spec(bn),
            pl.BlockSpec((bn, 1), lambda ii: (ii, 0)),
            pl.BlockSpec((256, 256), lambda ii: (0, 0)),
            _row_block_specs(bn),
            _row_block_specs(bn),
        ],
        out_specs=[
            _row_block_specs(bn),
            _chunk_block_spec(bn),
        ],
        out_shape=[
            jax.ShapeDtypeStruct((n, 256), jnp.float32),
            jax.ShapeDtypeStruct((4, n, 64), jnp.float32),
        ],
    )(s4, pa4, a, w, x, h1)


def _gate_users(il, bl, ft, gw, b2, nrows, bn):
    def body(il_ref, bl_ref, ft_ref, gw_ref, b_ref, o_ref):
        ilv = il_ref[...]
        blv = bl_ref[...]
        m = jnp.concatenate([ilv, blv, ft_ref[...]], axis=1)
        g = jnp.dot(m, gw_ref[...], preferred_element_type=jnp.float32)
        g = g + b_ref[...][0:1, :]
        w0 = 1.0 / (1.0 + jnp.exp(g[:, 1:2] - g[:, 0:1]))
        o_ref[...] = w0 * ilv + (1.0 - w0) * blv

    return pl.pallas_call(
        body,
        grid=(nrows // bn,),
        in_specs=[
            _row_block_specs(bn),
            _row_block_specs(bn),
            _row_block_specs(bn),
            pl.BlockSpec((768, 2), lambda ii: (0, 0)),
            pl.BlockSpec((1, 2), lambda ii: (0, 0)),
        ],
        out_specs=_row_block_specs(bn),
        out_shape=jax.ShapeDtypeStruct((nrows, 256), jnp.float32),
    )(il, bl, ft, gw, b2)


def _gate_bundles(il4, cnt, bl, ft, gw, b2, nrows, u, bn):
    def body(il_ref, c_ref, bl_ref, ft_ref, gw_ref, b_ref, o_ref):
        inv = 1.0 / (c_ref[...] + _EPS)
        ilv = jnp.concatenate(
            [il_ref[c] for c in range(4)], axis=1) * inv
        blv = bl_ref[...]
        m = jnp.concatenate([ilv, blv, ft_ref[...]], axis=1)
        g = jnp.dot(m, gw_ref[...], preferred_element_type=jnp.float32)
        g = g + b_ref[...][0:1, :]
        w0 = 1.0 / (1.0 + jnp.exp(g[:, 1:2] - g[:, 0:1]))
        o_ref[...] = w0 * ilv + (1.0 - w0) * blv

    uoff = u // bn
    return pl.pallas_call(
        body,
        grid=(nrows // bn,),
        in_specs=[
            _chunk_block_spec(bn),
            pl.BlockSpec((bn, 1), lambda ii: (ii, 0)),
            pl.BlockSpec((bn, 256), lambda ii: (ii + uoff, 0)),
            _row_block_specs(bn),
            pl.BlockSpec((768, 2), lambda ii: (0, 0)),
            pl.BlockSpec((1, 2), lambda ii: (0, 0)),
        ],
        out_specs=_row_block_specs(bn),
        out_shape=jax.ShapeDtypeStruct((nrows, 256), jnp.float32),
    )(il4, cnt, bl, ft, gw, b2)


def kernel(ui_edge_index, ub_edge_index, bi_edge_index, users_feature,
           items_feature, bundles_feature, W1_item, W2_item, W1_bundle,
           W2_bundle, gate_W, gate_b):
    u = users_feature.shape[0]
    i = items_feature.shape[0]
    b = bundles_feature.shape[0]
    n = u + i
    e = ui_edge_index.shape[1]

    ui0 = ui_edge_index[0].astype(jnp.int32)
    ui1 = ui_edge_index[1].astype(jnp.int32)
    ub0 = ub_edge_index[0].astype(jnp.int32)
    ub1 = ub_edge_index[1].astype(jnp.int32)
    bi0 = bi_edge_index[0].astype(jnp.int32)
    bi1 = bi_edge_index[1].astype(jnp.int32)

    epadh = _ceil_to(e, _NS * _LB * 8)
    nbt = epadh // _LB
    half = u
    coffs2 = (jnp.arange(4, dtype=jnp.int32) * n)[:, None, None]
    empty = jnp.zeros((0,), jnp.int32)
    spread = jnp.arange(epadh, dtype=jnp.int32) % n

    def padl(src, dl):
        ne = src.shape[0]
        pad = jnp.arange(epadh - ne, dtype=jnp.int32)
        return (jnp.concatenate([src, pad % n]),
                jnp.concatenate([dl, half + pad % 64]))

    def edges2(src0, dl0, src1, dl1):
        s0, d0 = padl(src0, dl0)
        s1, d1 = padl(src1, dl1)
        src2 = jnp.stack([s0, s1])
        d2 = jnp.stack([d0, d1])
        dst4 = jnp.broadcast_to(d2[None], (4, 2, epadh)).reshape(
            4, 2, nbt, _LB)
        src43 = (src2[None] + coffs2).reshape(4, 2, nbt, _LB)
        return src43, dst4

    trash = half + spread % 64
    hist_d = [
        (padl(spread[:e], ui0)[1], padl(spread[:e], ui1)[1]),
        (padl(spread[:e], ub0)[1], padl(spread[:e], ub1)[1]),
        (padl(spread[:e], bi0)[1], trash),
        (trash, trash),
    ]
    s4_h = (jnp.broadcast_to(spread[None, None], (4, 2, epadh)) +
            coffs2).reshape(4, 2, nbt, _LB)
    d_h = jnp.stack([jnp.stack(list(p)) for p in hist_d]).reshape(
        4, 2, nbt, _LB)

    s4_ui, d_ui = edges2(ui1 + u, ui0, ui0, ui1)
    s4_ub, d_ub = edges2(ub1 + u, ub0, ub0, ub1)
    s4_bi, d_bi = edges2(bi1 + u, bi0, empty, empty)

    src4_stack = jnp.stack([s4_h, s4_ui, s4_ui, s4_ub, s4_ub, s4_bi])
    dst_stack = jnp.stack([d_h, d_ui, d_ui, d_ub, d_ub, d_bi])

    scat = _make_scatter(epadh, n, half)
    bn = 1000

    x_ui = jnp.concatenate([users_feature, items_feature], axis=0)
    x_ub = jnp.concatenate([users_feature, bundles_feature], axis=0)
    ones_tbl = jnp.ones((4 * n, 64), jnp.float32)

    zn1 = jnp.zeros((n, 1), jnp.float32)
    znd = jnp.zeros((n, 256), jnp.float32)
    zn4 = jnp.zeros((4, n, 64), jnp.float32)
    stash0 = (zn1, zn1, zn1, zn4, zn4, znd, znd, znd, znd, zn4, zn4)

    def _upd(st, **kw):
        names = ["a_ui", "a_ub", "cnt_bi", "xa4_ui", "xa4_ub", "h1_ui",
                 "h1_ub", "out_ui", "out_ub", "out4_ui", "pooled4"]
        lst = list(st)
        for k, v in kw.items():
            lst[names.index(k)] = v
        return tuple(lst)

    def br_hist(tbl, out4, st):
        a_ui, xa4_ui = _scale_a(x_ui, out4[0, :, 0:1], bn)
        a_ub, xa4_ub = _scale_a(x_ub, out4[1, :, 0:1], bn)
        return xa4_ui.reshape(4 * n, 64), _upd(
            st, a_ui=a_ui, a_ub=a_ub, cnt_bi=out4[2, :, 0:1],
            xa4_ui=xa4_ui, xa4_ub=xa4_ub)

    def br_s1_ui(tbl, out4, st):
        h1, ha4 = _layer1(out4, st[3], st[0], W1_item, bn)
        return ha4.reshape(4 * n, 64), _upd(st, h1_ui=h1)

    def br_s2_ui(tbl, out4, st):
        out, o4 = _layer2(out4, tbl.reshape(4, n, 64), st[0], W2_item, x_ui,
                          st[5], bn)
        return st[4].reshape(4 * n, 64), _upd(st, out_ui=out, out4_ui=o4)

    def br_s1_ub(tbl, out4, st):
        h1, ha4 = _layer1(out4, st[4], st[1], W1_bundle, bn)
        return ha4.reshape(4 * n, 64), _upd(st, h1_ub=h1)

    def br_s2_ub(tbl, out4, st):
        out, _o4 = _layer2(out4, tbl.reshape(4, n, 64), st[1], W2_bundle,
                           x_ub, st[6], bn)
        return st[9].reshape(4 * n, 64), _upd(st, out_ub=out)

    def br_pool(tbl, out4, st):
        return tbl, _upd(st, pooled4=out4)

    branches = [br_hist, br_s1_ui, br_s2_ui, br_s1_ub, br_s2_ub, br_pool]

    def body(carry, xs):
        tbl, st = carry
        src4, dst, pid = xs
        out4 = scat(tbl, src4, dst)
        tbl2, st2 = lax.switch(pid, branches, tbl, out4, st)
        return (tbl2, st2), None

    (_, stash), _ = lax.scan(
        body, (ones_tbl, stash0),
        (src4_stack, dst_stack, jnp.arange(6, dtype=jnp.int32)))

    b2 = gate_b.reshape(1, 2)
    users_out = _gate_users(stash[7], stash[8], users_feature, gate_W, b2,
                            u, bn)
    bundles_out = _gate_bundles(stash[10], stash[2], stash[8],
                                bundles_feature, gate_W, b2, b, u, bn)

    return jnp.concatenate([users_out, bundles_out], axis=0)
